# Initial kernel scaffold; baseline (speedup 1.0000x reference)
import jax
import jax.numpy as jnp
from jax.experimental import pallas as pl


def kernel(scores, gold_target, transitions):
    raise NotImplementedError("write your pallas kernel here")


# speedup vs baseline: 21.6489x; 21.6489x over previous
import jax
import jax.numpy as jnp
from jax.experimental import pallas as pl
from jax.experimental.pallas import tpu as pltpu

_K = 64
_START = 61
_END = 63


def _crf_fwd_kernel(scores_t_ref, gold_ref, t_ref, out_ref, e_scr):
    T = t_ref[:]
    expT = jnp.exp(T)
    s_all = scores_t_ref[:]
    Ln, Bn, Kn = s_all.shape
    ms = jnp.max(s_all, axis=2, keepdims=True)
    e_scr[:] = jnp.exp(s_all - ms)

    c0 = jnp.max(T[_START, :])
    F0 = jnp.broadcast_to(jnp.exp(T[_START, :] - c0)[None, :], (Bn, Kn))
    loga0 = jnp.zeros((Bn, 1), dtype=jnp.float32)

    def body(t, carry):
        F, loga = carry
        G = F * e_scr[t]
        H = jax.lax.dot_general(
            G, expT, (((1,), (0,)), ((), ())),
            preferred_element_type=jnp.float32,
        )
        h = jnp.max(H, axis=1, keepdims=True)
        F_new = H * (1.0 / h)
        return F_new, loga + jnp.log(h)

    F_fin, loga = jax.lax.fori_loop(0, Ln, body, (F0, loga0))

    sum_ms = jnp.sum(ms[:, :, 0], axis=0)
    fs_end = c0 + sum_ms + loga[:, 0] + jnp.log(F_fin[:, _END])
    forscores = jnp.sum(fs_end)

    sum_s0 = jnp.sum(s_all[:, :, 0])
    g = gold_ref[:]
    onehot = (g[:, :, None] ==
              jax.lax.broadcasted_iota(jnp.int32, (1, 1, Kn), 2))
    cnt = jnp.sum(onehot.astype(jnp.float32), axis=(0, 1))
    tg = Bn * T[0, _START] + sum_s0 + jnp.sum(cnt * T[0, :])

    loss = (forscores - tg) / Bn
    out_ref[:, :] = jnp.broadcast_to(loss, (1, 1))


def kernel(scores, gold_target, transitions):
    B, L, K = scores.shape
    scores_t = jnp.transpose(scores, (1, 0, 2))
    out = pl.pallas_call(
        _crf_fwd_kernel,
        out_shape=jax.ShapeDtypeStruct((1, 1), jnp.float32),
        scratch_shapes=[pltpu.VMEM((L, B, K), jnp.float32)],
    )(scores_t, gold_target, transitions)
    return out[0, 0]


# speedup vs baseline: 31.9380x; 1.4753x over previous
import jax
import jax.numpy as jnp
from jax.experimental import pallas as pl
from jax.experimental.pallas import tpu as pltpu

_K = 64
_START = 61
_END = 63


def _crf_fwd_kernel(scores_t_ref, gold_ref, t_ref, out_ref, e_scr):
    T = t_ref[:]
    Kn = T.shape[0]
    s_all = scores_t_ref[:]
    Ln, Bn, _ = s_all.shape
    Hn = Bn // 2

    expT = jnp.exp(T)
    z = jnp.zeros((Kn, Kn), jnp.float32)
    X2 = jnp.concatenate(
        [jnp.concatenate([expT, z], axis=1),
         jnp.concatenate([z, expT], axis=1)], axis=0
    ).astype(jnp.bfloat16)

    e_scr[:] = jnp.concatenate(
        [jnp.exp(s_all[:, :Hn, :]), jnp.exp(s_all[:, Hn:, :])], axis=2
    ).astype(jnp.bfloat16)

    c0 = jnp.max(T[_START, :])
    v0 = jnp.exp(T[_START, :] - c0)
    row_one = (jax.lax.broadcasted_iota(jnp.int32, (Hn, 1), 0)
               .astype(jnp.float32) * 0.0) + 1.0
    ones = jnp.broadcast_to(row_one, (Hn, 2 * Kn))
    zer = ones - ones
    A0 = jnp.concatenate([v0, v0])[None, :] * ones

    def body(t, carry):
        A, r_apply, r_pend, loga = carry
        Bv = A * r_apply
        G = (Bv * e_scr[t].astype(jnp.float32)).astype(jnp.bfloat16)
        Anew = jax.lax.dot_general(
            G, X2, (((1,), (0,)), ((), ())),
            preferred_element_type=jnp.float32,
        )
        h = jnp.max(Bv, axis=1, keepdims=True)
        rnew = jnp.broadcast_to(1.0 / h, (Hn, 2 * Kn))
        return Anew, r_pend, rnew, loga - jnp.log(rnew)

    A, r_ap, r_pe, loga = jax.lax.fori_loop(
        0, Ln, body, (A0, ones, ones, zer))
    Bf = A * (r_ap * r_pe)
    fs_end = (2.0 * (c0 + loga[:, 0])
              + jnp.log(Bf[:, _END]) + jnp.log(Bf[:, Kn + _END]))
    forscores = jnp.sum(fs_end)

    sum_s0 = jnp.sum(s_all[:, :, 0])
    g = gold_ref[:]
    onehot = (g[:, :, None] ==
              jax.lax.broadcasted_iota(jnp.int32, (1, 1, Kn), 2))
    cnt = jnp.sum(onehot.astype(jnp.float32), axis=(0, 1))
    tg = Bn * T[0, _START] + sum_s0 + jnp.sum(cnt * T[0, :])

    loss = (forscores - tg) / Bn
    out_ref[:, :] = jnp.broadcast_to(loss, (1, 1))


def kernel(scores, gold_target, transitions):
    B, L, K = scores.shape
    scores_t = jnp.transpose(scores, (1, 0, 2))
    out = pl.pallas_call(
        _crf_fwd_kernel,
        out_shape=jax.ShapeDtypeStruct((1, 1), jnp.float32),
        scratch_shapes=[pltpu.VMEM((L, B // 2, 2 * K), jnp.bfloat16)],
    )(scores_t, gold_target, transitions)
    return out[0, 0]


# speedup vs baseline: 50.0103x; 1.5659x over previous
import jax
import jax.numpy as jnp
from jax.experimental import pallas as pl
from jax.experimental.pallas import tpu as pltpu

_K = 64
_START = 61
_END = 63
_PADW = 8


def _crf_fwd_kernel(scores_t_ref, gold_ref, t_ref, out_ref, ef_scr, eb_scr):
    T = t_ref[:]
    Kn = T.shape[0]
    s_all = scores_t_ref[:]
    Ln, Bn, _ = s_all.shape
    Wn = Kn + _PADW

    expT = jnp.exp(T)
    padc = jnp.zeros((Kn, _PADW - 1), jnp.float32)
    padr = jnp.zeros((_PADW, Wn), jnp.float32)

    def augment(M):
        rs = jnp.sum(M, axis=1, keepdims=True)
        return jnp.concatenate(
            [jnp.concatenate([M, rs, padc], axis=1), padr],
            axis=0).astype(jnp.bfloat16)

    XaF = augment(expT)
    XaB = augment(expT.T)

    exps = jnp.exp(s_all)
    zpad = jnp.zeros((Ln, Bn, _PADW), jnp.float32)
    opad = jnp.concatenate(
        [jnp.ones((Ln, Bn, 1), jnp.float32),
         jnp.zeros((Ln, Bn, _PADW - 1), jnp.float32)], axis=2)
    ef_scr[:] = jnp.concatenate([exps, zpad], axis=2).astype(jnp.bfloat16)
    eb_scr[:] = jnp.concatenate([exps, opad], axis=2).astype(jnp.bfloat16)

    c0 = jnp.max(T[_START, :])
    v0 = jnp.exp(T[_START, :] - c0)
    row_one = (jax.lax.broadcasted_iota(jnp.int32, (Bn, 1), 0)
               .astype(jnp.float32) * 0.0) + 1.0
    ones = jnp.broadcast_to(row_one, (Bn, Wn))
    zer = ones - ones
    a0row = jnp.concatenate(
        [v0, jnp.sum(v0)[None], jnp.zeros((_PADW - 1,), jnp.float32)])
    Af0 = a0row[None, :] * ones
    lane = jax.lax.broadcasted_iota(jnp.int32, (1, Wn), 1)
    u0row = ((lane == _END) | (lane == Kn)).astype(jnp.float32)
    Au0 = u0row * ones

    def body(t, carry):
        Af, rf, Au, ru, loga = carry
        Fv = Af * rf
        Uv = Au * ru
        hf = Fv[:, Kn:Kn + 1]
        hu = Uv[:, Kn:Kn + 1]
        rfn = jnp.broadcast_to(1.0 / hf, (Bn, Wn))
        run = jnp.broadcast_to(1.0 / hu, (Bn, Wn))
        Gf = (Fv * ef_scr[t].astype(jnp.float32)).astype(jnp.bfloat16)
        Afn = jnp.concatenate([
            jax.lax.dot_general(Gf[:16], XaF, (((1,), (0,)), ((), ())),
                                preferred_element_type=jnp.float32),
            jax.lax.dot_general(Gf[16:], XaF, (((1,), (0,)), ((), ())),
                                preferred_element_type=jnp.float32)], axis=0)
        Gu = Uv.astype(jnp.bfloat16)
        Aun = jnp.concatenate([
            jax.lax.dot_general(Gu[:16], XaB, (((1,), (0,)), ((), ())),
                                preferred_element_type=jnp.float32),
            jax.lax.dot_general(Gu[16:], XaB, (((1,), (0,)), ((), ())),
                                preferred_element_type=jnp.float32)], axis=0
        ) * eb_scr[Ln - 1 - t].astype(jnp.float32)
        return Afn, rfn, Aun, run, loga - jnp.log(rfn) - jnp.log(run)

    Af, rf, Au, ru, loga = jax.lax.fori_loop(
        0, Ln // 2, body, (Af0, ones, Au0, ones, zer))
    Ff = Af * rf
    Uf = Au * ru
    dot_mid = jnp.sum((Ff * Uf)[:, :Kn], axis=1)
    fs_end = c0 + loga[:, 0] + jnp.log(dot_mid)
    forscores = jnp.sum(fs_end)

    sum_s0 = jnp.sum(s_all[:, :, 0])
    g = gold_ref[:]
    onehot = (g[:, :, None] ==
              jax.lax.broadcasted_iota(jnp.int32, (1, 1, Kn), 2))
    cnt = jnp.sum(onehot.astype(jnp.float32), axis=(0, 1))
    tg = Bn * T[0, _START] + sum_s0 + jnp.sum(cnt * T[0, :])

    loss = (forscores - tg) / Bn
    out_ref[:, :] = jnp.broadcast_to(loss, (1, 1))


def kernel(scores, gold_target, transitions):
    B, L, K = scores.shape
    scores_t = jnp.transpose(scores, (1, 0, 2))
    out = pl.pallas_call(
        _crf_fwd_kernel,
        out_shape=jax.ShapeDtypeStruct((1, 1), jnp.float32),
        scratch_shapes=[pltpu.VMEM((L, B, K + _PADW), jnp.bfloat16),
                        pltpu.VMEM((L, B, K + _PADW), jnp.bfloat16)],
    )(scores_t, gold_target, transitions)
    return out[0, 0]


# speedup vs baseline: 51.2663x; 1.0251x over previous
import jax
import jax.numpy as jnp
from jax.experimental import pallas as pl
from jax.experimental.pallas import tpu as pltpu

_K = 64
_START = 61
_END = 63
_PADW = 8


def _crf_fwd_kernel(scores_t_ref, gold_ref, t_ref, out_ref, ef_scr, eb_scr):
    T = t_ref[:]
    Kn = T.shape[0]
    s_all = jnp.transpose(scores_t_ref[:], (1, 0, 2))
    Ln, Bn, _ = s_all.shape
    Wn = Kn + _PADW

    expT = jnp.exp(T)
    padc = jnp.zeros((Kn, _PADW - 1), jnp.float32)
    padr = jnp.zeros((_PADW, Wn), jnp.float32)

    def augment(M):
        rs = jnp.sum(M, axis=1, keepdims=True)
        return jnp.concatenate(
            [jnp.concatenate([M, rs, padc], axis=1), padr],
            axis=0).astype(jnp.bfloat16)

    XaF = augment(expT)
    XaB = augment(expT.T)

    exps = jnp.exp(s_all)
    zpad = jnp.zeros((Ln, Bn, _PADW), jnp.float32)
    opad = jnp.concatenate(
        [jnp.ones((Ln, Bn, 1), jnp.float32),
         jnp.zeros((Ln, Bn, _PADW - 1), jnp.float32)], axis=2)
    ef_scr[:] = jnp.concatenate([exps, zpad], axis=2).astype(jnp.bfloat16)
    eb_scr[:] = jnp.concatenate([exps, opad], axis=2).astype(jnp.bfloat16)

    c0 = jnp.max(T[_START, :])
    v0 = jnp.exp(T[_START, :] - c0)
    row_one = (jax.lax.broadcasted_iota(jnp.int32, (Bn, 1), 0)
               .astype(jnp.float32) * 0.0) + 1.0
    ones = jnp.broadcast_to(row_one, (Bn, Wn))
    zer = ones - ones
    a0row = jnp.concatenate(
        [v0, jnp.sum(v0)[None], jnp.zeros((_PADW - 1,), jnp.float32)])
    Af0 = a0row[None, :] * ones
    lane = jax.lax.broadcasted_iota(jnp.int32, (1, Wn), 1)
    u0row = ((lane == _END) | (lane == Kn)).astype(jnp.float32)
    Au0 = u0row * ones

    def body(t, carry):
        Af, rf, Au, ru, loga = carry
        Fv = Af * rf
        Uv = Au * ru
        hf = Fv[:, Kn:Kn + 1]
        hu = Uv[:, Kn:Kn + 1]
        rfn = jnp.broadcast_to(1.0 / hf, (Bn, Wn))
        run = jnp.broadcast_to(1.0 / hu, (Bn, Wn))
        Gf = (Fv * ef_scr[t].astype(jnp.float32)).astype(jnp.bfloat16)
        Afn = jnp.concatenate([
            jax.lax.dot_general(Gf[:16], XaF, (((1,), (0,)), ((), ())),
                                preferred_element_type=jnp.float32),
            jax.lax.dot_general(Gf[16:], XaF, (((1,), (0,)), ((), ())),
                                preferred_element_type=jnp.float32)], axis=0)
        Gu = Uv.astype(jnp.bfloat16)
        Aun = jnp.concatenate([
            jax.lax.dot_general(Gu[:16], XaB, (((1,), (0,)), ((), ())),
                                preferred_element_type=jnp.float32),
            jax.lax.dot_general(Gu[16:], XaB, (((1,), (0,)), ((), ())),
                                preferred_element_type=jnp.float32)], axis=0
        ) * eb_scr[Ln - 1 - t].astype(jnp.float32)
        return Afn, rfn, Aun, run, loga - jnp.log(rfn) - jnp.log(run)

    Af, rf, Au, ru, loga = jax.lax.fori_loop(
        0, Ln // 2, body, (Af0, ones, Au0, ones, zer))
    Ff = Af * rf
    Uf = Au * ru
    dot_mid = jnp.sum((Ff * Uf)[:, :Kn], axis=1)
    fs_end = c0 + loga[:, 0] + jnp.log(dot_mid)
    forscores = jnp.sum(fs_end)

    sum_s0 = jnp.sum(s_all[:, :, 0])
    g = gold_ref[:]
    onehot = (g[:, :, None] ==
              jax.lax.broadcasted_iota(jnp.int32, (1, 1, Kn), 2))
    cnt = jnp.sum(onehot.astype(jnp.float32), axis=(0, 1))
    tg = Bn * T[0, _START] + sum_s0 + jnp.sum(cnt * T[0, :])

    loss = (forscores - tg) / Bn
    out_ref[:, :] = jnp.broadcast_to(loss, (1, 1))


def kernel(scores, gold_target, transitions):
    B, L, K = scores.shape
    out = pl.pallas_call(
        _crf_fwd_kernel,
        out_shape=jax.ShapeDtypeStruct((1, 1), jnp.float32),
        scratch_shapes=[pltpu.VMEM((L, B, K + _PADW), jnp.bfloat16),
                        pltpu.VMEM((L, B, K + _PADW), jnp.bfloat16)],
    )(scores, gold_target, transitions)
    return out[0, 0]


# speedup vs baseline: 53.2713x; 1.0391x over previous
import jax
import jax.numpy as jnp
from jax.experimental import pallas as pl
from jax.experimental.pallas import tpu as pltpu

_K = 64
_START = 61
_END = 63
_PADW = 8


def _crf_fwd_kernel(scores_t_ref, gold_ref, t_ref, out_ref, ef_scr, eb_scr):
    T = t_ref[:]
    Kn = T.shape[0]
    s_all = jnp.transpose(scores_t_ref[:], (1, 0, 2))
    Ln, Bn, _ = s_all.shape
    Wn = Kn + _PADW

    expT = jnp.exp(T)
    padc = jnp.zeros((Kn, _PADW - 1), jnp.float32)
    padr = jnp.zeros((_PADW, Wn), jnp.float32)

    def augment(M):
        rs = jnp.sum(M, axis=1, keepdims=True)
        return jnp.concatenate(
            [jnp.concatenate([M, rs, padc], axis=1), padr],
            axis=0).astype(jnp.bfloat16)

    XaF = augment(expT)
    XaB = augment(expT.T)

    exps = jnp.exp(s_all)
    zpad = jnp.zeros((Ln, Bn, _PADW), jnp.float32)
    opad = jnp.concatenate(
        [jnp.ones((Ln, Bn, 1), jnp.float32),
         jnp.zeros((Ln, Bn, _PADW - 1), jnp.float32)], axis=2)
    ef_scr[:] = jnp.concatenate([exps, zpad], axis=2).astype(jnp.bfloat16)
    eb_scr[:] = jnp.concatenate([exps, opad], axis=2).astype(jnp.bfloat16)

    c0 = jnp.max(T[_START, :])
    v0 = jnp.exp(T[_START, :] - c0)
    row_one = (jax.lax.broadcasted_iota(jnp.int32, (Bn, 1), 0)
               .astype(jnp.float32) * 0.0) + 1.0
    ones = jnp.broadcast_to(row_one, (Bn, Wn))
    zer = ones - ones
    a0row = jnp.concatenate(
        [v0, jnp.sum(v0)[None], jnp.zeros((_PADW - 1,), jnp.float32)])
    Af0 = a0row[None, :] * ones
    lane = jax.lax.broadcasted_iota(jnp.int32, (1, Wn), 1)
    u0row = ((lane == _END) | (lane == Kn)).astype(jnp.float32)
    Au0 = u0row * ones

    def body(t, carry):
        Af, rf, Au, ru, loga = carry
        Fv = Af * rf
        Uv = Au * ru
        hf = Fv[:, Kn:Kn + 1]
        hu = Uv[:, Kn:Kn + 1]
        rfn = jnp.broadcast_to(1.0 / hf, (Bn, Wn))
        run = jnp.broadcast_to(1.0 / hu, (Bn, Wn))
        Gf = (Fv * ef_scr[t].astype(jnp.float32)).astype(jnp.bfloat16)
        Afn = jnp.concatenate([
            jax.lax.dot_general(Gf[:16], XaF, (((1,), (0,)), ((), ())),
                                preferred_element_type=jnp.float32),
            jax.lax.dot_general(Gf[16:], XaF, (((1,), (0,)), ((), ())),
                                preferred_element_type=jnp.float32)], axis=0)
        Gu = Uv.astype(jnp.bfloat16)
        Aun = jnp.concatenate([
            jax.lax.dot_general(Gu[:16], XaB, (((1,), (0,)), ((), ())),
                                preferred_element_type=jnp.float32),
            jax.lax.dot_general(Gu[16:], XaB, (((1,), (0,)), ((), ())),
                                preferred_element_type=jnp.float32)], axis=0
        ) * eb_scr[Ln - 1 - t].astype(jnp.float32)
        return Afn, rfn, Aun, run, loga - jnp.log(rfn) - jnp.log(run)

    def body2(i, carry):
        return body(2 * i + 1, body(2 * i, carry))

    Af, rf, Au, ru, loga = jax.lax.fori_loop(
        0, Ln // 4, body2, (Af0, ones, Au0, ones, zer))
    Ff = Af * rf
    Uf = Au * ru
    dot_mid = jnp.sum((Ff * Uf)[:, :Kn], axis=1)
    fs_end = c0 + loga[:, 0] + jnp.log(dot_mid)
    forscores = jnp.sum(fs_end)

    sum_s0 = jnp.sum(s_all[:, :, 0])
    g = gold_ref[:]
    onehot = (g[:, :, None] ==
              jax.lax.broadcasted_iota(jnp.int32, (1, 1, Kn), 2))
    cnt = jnp.sum(onehot.astype(jnp.float32), axis=(0, 1))
    tg = Bn * T[0, _START] + sum_s0 + jnp.sum(cnt * T[0, :])

    loss = (forscores - tg) / Bn
    out_ref[:, :] = jnp.broadcast_to(loss, (1, 1))


def kernel(scores, gold_target, transitions):
    B, L, K = scores.shape
    out = pl.pallas_call(
        _crf_fwd_kernel,
        out_shape=jax.ShapeDtypeStruct((1, 1), jnp.float32),
        scratch_shapes=[pltpu.VMEM((L, B, K + _PADW), jnp.bfloat16),
                        pltpu.VMEM((L, B, K + _PADW), jnp.bfloat16)],
    )(scores, gold_target, transitions)
    return out[0, 0]
